# baseline (speedup 1.0000x reference)
import jax
import jax.numpy as jnp
from jax.experimental import pallas as pl

RANK = 10
NCF = 32
ORD = 5
NUM_IT = 5
R_MIN = 1.0
R_RANGE = 4.0


def _score_body(wu_ref, hi_ref, out_ref):
    s = jnp.sum(wu_ref[:] * hi_ref[:], axis=1, keepdims=True)
    out_ref[:] = R_MIN + R_RANGE * jax.nn.sigmoid(s)


def _make_mult(edge_index, n):
    src = edge_index[0]
    dst = edge_index[1]
    deg = jax.ops.segment_sum(jnp.ones(src.shape[0], jnp.float32), dst, num_segments=n)
    dinv = 1.0 / jnp.sqrt(jnp.maximum(deg, 1.0))
    w = dinv[src] * dinv[dst]

    def mult(X):
        return -jax.ops.segment_sum(w[:, None] * X[src], dst, num_segments=n)

    return mult


def _cheb(mult, X):
    t0 = X
    t1 = mult(X)
    ts = [t0, t1]
    for _ in range(ORD - 2):
        ts.append(2.0 * mult(ts[-1]) - ts[-2])
    return jnp.concatenate(ts, axis=1)


def kernel(W0, H0, params, edge_index_user, edge_index_item, user_id, item_id):
    mult_u = _make_mult(edge_index_user, W0.shape[0])
    mult_m = _make_mult(edge_index_item, H0.shape[0])
    p = params
    sig = jax.nn.sigmoid
    W, H = W0, H0
    h_u = jnp.zeros((W0.shape[0], NCF), jnp.float32)
    c_u = jnp.zeros_like(h_u)
    h_m = jnp.zeros((H0.shape[0], NCF), jnp.float32)
    c_m = jnp.zeros_like(h_m)
    for _ in range(NUM_IT):
        x_u = sig(_cheb(mult_u, W) @ p['W_conv_W'] + p['b_conv_W'])
        f = sig(x_u @ p['W_f_u'] + h_u @ p['U_f_u'] + p['b_f_u'])
        i_ = sig(x_u @ p['W_i_u'] + h_u @ p['U_i_u'] + p['b_i_u'])
        o = sig(x_u @ p['W_o_u'] + h_u @ p['U_o_u'] + p['b_o_u'])
        ct = jnp.tanh(x_u @ p['W_c_u'] + h_u @ p['U_c_u'] + p['b_c_u'])
        c_u = f * c_u + i_ * ct
        h_u = o * jnp.tanh(c_u)
        W = W + jnp.tanh(h_u @ p['W_out_u'] + p['b_out_u'])
        x_m = sig(_cheb(mult_m, H) @ p['W_conv_H'] + p['b_conv_H'])
        f = sig(x_m @ p['W_f_m'] + h_m @ p['U_f_m'] + p['b_f_m'])
        i_ = sig(x_m @ p['W_i_m'] + h_m @ p['U_i_m'] + p['b_i_m'])
        o = sig(x_m @ p['W_o_m'] + h_m @ p['U_o_m'] + p['b_o_m'])
        ct = jnp.tanh(x_m @ p['W_c_m'] + h_m @ p['U_c_m'] + p['b_c_m'])
        c_m = f * c_m + i_ * ct
        h_m = o * jnp.tanh(c_m)
        H = H + jnp.tanh(h_m @ p['W_out_m'] + p['b_out_m'])
    wu = jnp.take(W, user_id, axis=0)
    hi = jnp.take(H, item_id, axis=0)
    B = user_id.shape[0]
    score = pl.pallas_call(
        _score_body,
        out_shape=jax.ShapeDtypeStruct((B, 1), jnp.float32),
    )(wu, hi)
    return score[:, 0]


# speedup vs baseline: 13.4708x; 13.4708x over previous
import jax
import jax.numpy as jnp
from jax import lax
from jax.experimental import pallas as pl
from jax.experimental.pallas import tpu as pltpu
from jax.experimental.pallas import tpu_sc as plsc

N = 50000
RANK = 10
NCF = 32
ORD = 5
NUM_IT = 5
R_MIN = 1.0
R_RANGE = 4.0
B = 16384

F = 16
NP = 50176
TR = NP // 16
E = 800000
GPH = 200
EP = 32 * GPH * 128
CG = 10
NCHUNK = GPH // CG
FLAT = (NP * F // 128, 128)
FB = FLAT[0] // 8
DR = NP // 16

_mesh = plsc.VectorSubcoreMesh(core_axis_name="c", subcore_axis_name="s")


def _spmm_body(xs_u, xs_m, src_u, dst_u, src_m, dst_m, zeros,
               acc_u, acc_m, accs_u, accs_m):
    pl.run_scoped(
        lambda src_v, dst_v, rows_v, gsem, ssem: _spmm_inner(
            xs_u, xs_m, src_u, dst_u, src_m, dst_m, zeros,
            acc_u, acc_m, accs_u, accs_m,
            src_v, dst_v, rows_v, gsem, ssem),
        pltpu.VMEM((CG, 128), jnp.int32),
        pltpu.VMEM((CG, 128), jnp.int32),
        pltpu.VMEM((CG * 128, F), jnp.float32),
        pltpu.SemaphoreType.DMA,
        pltpu.SemaphoreType.DMA,
    )


def _spmm_inner(xs_u, xs_m, src_u, dst_u, src_m, dst_m, zeros,
                acc_u, acc_m, accs_u, accs_m, src_v, dst_v, rows_v,
                gsem, ssem):
    c = lax.axis_index("c")
    s = lax.axis_index("s")
    rbase = s * TR
    pltpu.sync_copy(zeros.at[pl.ds(rbase, TR)], accs_u.at[pl.ds(rbase, TR)])
    pltpu.sync_copy(zeros.at[pl.ds(rbase, TR)], accs_m.at[pl.ds(rbase, TR)])
    plsc.subcore_barrier()
    gbase = (c * 16 + s) * GPH

    def run(xs, src2, dst2, accs):
        def chunk(ci, carry):
            g0 = gbase + ci * CG
            pltpu.sync_copy(src2.at[pl.ds(g0, CG)], src_v)
            pltpu.sync_copy(dst2.at[pl.ds(g0, CG)], dst_v)
            cps = [pltpu.async_copy(xs.at[src_v.at[g]],
                                    rows_v.at[pl.ds(g * 128, 128)], gsem)
                   for g in range(CG)]
            for cp in cps:
                cp.wait()
            scs = [pltpu.async_copy(rows_v.at[pl.ds(g * 128, 128)],
                                    accs.at[dst_v.at[g]], ssem, add=True)
                   for g in range(CG)]
            for sc in scs:
                sc.wait()
            return carry
        lax.fori_loop(0, NCHUNK, chunk, 0)

    run(xs_u, src_u, dst_u, accs_u)
    run(xs_m, src_m, dst_m, accs_m)
    plsc.subcore_barrier()
    pltpu.sync_copy(accs_u.at[pl.ds(rbase, TR)],
                    acc_u.at[c, pl.ds(rbase, TR)])
    pltpu.sync_copy(accs_m.at[pl.ds(rbase, TR)],
                    acc_m.at[c, pl.ds(rbase, TR)])


_spmm = pl.kernel(
    _spmm_body,
    out_type=(jax.ShapeDtypeStruct((2, NP, F), jnp.float32),
              jax.ShapeDtypeStruct((2, NP, F), jnp.float32)),
    mesh=_mesh,
    scratch_types=[
        pltpu.VMEM_SHARED((NP, F), jnp.float32),
        pltpu.VMEM_SHARED((NP, F), jnp.float32),
    ],
    compiler_params=pltpu.CompilerParams(use_tc_tiling_on_sc=False),
)


def _gather_body(w16, h16, uid2, iid2, wu, hi):
    pl.run_scoped(
        lambda idx_v, rows_v, gsem: _gather_inner(
            w16, h16, uid2, iid2, wu, hi, idx_v, rows_v, gsem),
        pltpu.VMEM((4, 128), jnp.int32),
        pltpu.VMEM((512, F), jnp.float32),
        pltpu.SemaphoreType.DMA,
    )


def _gather_inner(w16, h16, uid2, iid2, wu, hi, idx_v, rows_v, gsem):
    c = lax.axis_index("c")
    s = lax.axis_index("s")
    wid = c * 16 + s

    def run(tab, idx2, out):
        pltpu.sync_copy(idx2.at[pl.ds(wid * 4, 4)], idx_v)
        cps = [pltpu.async_copy(tab.at[idx_v.at[g]],
                                rows_v.at[pl.ds(g * 128, 128)], gsem)
               for g in range(4)]
        for cp in cps:
            cp.wait()
        pltpu.sync_copy(rows_v, out.at[pl.ds(wid * 512, 512)])

    run(w16, uid2, wu)
    run(h16, iid2, hi)


_gather = pl.kernel(
    _gather_body,
    out_type=(jax.ShapeDtypeStruct((B, F), jnp.float32),
              jax.ShapeDtypeStruct((B, F), jnp.float32)),
    mesh=_mesh,
    compiler_params=pltpu.CompilerParams(use_tc_tiling_on_sc=False),
)


def _prep_body(du, dm, wp, hp, dvu, xsu, dvm, xsm):
    dd = du[:]
    a = lax.rsqrt(jnp.maximum(dd[0] + dd[1], 1.0))
    dvu[:] = a
    xsu[:] = a * wp[:]
    dd = dm[:]
    b = lax.rsqrt(jnp.maximum(dd[0] + dd[1], 1.0))
    dvm[:] = b
    xsm[:] = b * hp[:]


def _glue1_body(au, dvu, am, dvm, tu, xsu, tm, xsm):
    a = au[:]
    t = -(dvu[:] * (a[0] + a[1]))
    tu[:] = t
    xsu[:] = dvu[:] * t
    a = am[:]
    t2 = -(dvm[:] * (a[0] + a[1]))
    tm[:] = t2
    xsm[:] = dvm[:] * t2


def _glue2_body(au, tpu_, dvu, am, tpm, dvm, tu, xsu, tm, xsm):
    a = au[:]
    t = -2.0 * (dvu[:] * (a[0] + a[1])) - tpu_[:]
    tu[:] = t
    xsu[:] = dvu[:] * t
    a = am[:]
    t2 = -2.0 * (dvm[:] * (a[0] + a[1])) - tpm[:]
    tm[:] = t2
    xsm[:] = dvm[:] * t2


_ACC_SHAPE = (2,) + FLAT


def _ew(body, in_kinds, n_out):
    bs = pl.BlockSpec((FB, 128), lambda i: (i, 0))
    bs2 = pl.BlockSpec((2, FB, 128), lambda i: (0, i, 0))
    return pl.pallas_call(
        body,
        out_shape=tuple(jax.ShapeDtypeStruct(FLAT, jnp.float32)
                        for _ in range(n_out)),
        grid=(8,),
        in_specs=[bs2 if k == 2 else bs for k in in_kinds],
        out_specs=[bs] * n_out,
    )


_prep = _ew(_prep_body, (2, 2, 1, 1), 4)
_glue1 = _ew(_glue1_body, (2, 1, 2, 1), 4)
_glue2 = _ew(_glue2_body, (2, 1, 1, 2, 1, 1), 4)


def _dense_body(t0, t1, t2, t3, t4, h, c, wm, dv,
                wc, bc, wf, uf, bf, wi, ui, bi, wo, uo, bo, wg, ug, bg,
                wout, bout, hn_o, cn_o, wn_o, xs_o):
    sig = jax.nn.sigmoid
    wcv = wc[:]
    z = (t0[:] @ wcv[0 * F:1 * F] + t1[:] @ wcv[1 * F:2 * F]
         + t2[:] @ wcv[2 * F:3 * F] + t3[:] @ wcv[3 * F:4 * F]
         + t4[:] @ wcv[4 * F:5 * F])
    x = sig(z + bc[:])
    hv = h[:]
    cv = c[:]
    f = sig(x @ wf[:] + hv @ uf[:] + bf[:])
    i_ = sig(x @ wi[:] + hv @ ui[:] + bi[:])
    o = sig(x @ wo[:] + hv @ uo[:] + bo[:])
    ct = jnp.tanh(x @ wg[:] + hv @ ug[:] + bg[:])
    cn = f * cv + i_ * ct
    hn = o * jnp.tanh(cn)
    wn = wm[:] + jnp.tanh(hn @ wout[:] + bout[:])
    hn_o[:] = hn
    cn_o[:] = cn
    wn_o[:] = wn
    xs_o[:] = dv[:] * wn


def _mk_dense():
    bF = pl.BlockSpec((DR, F), lambda i: (i, 0))
    bC = pl.BlockSpec((DR, NCF), lambda i: (i, 0))
    full = lambda shape: pl.BlockSpec(shape, lambda i: (0, 0))
    in_specs = ([bF] * 5 + [bC, bC, bF, bF]
                + [full((ORD * F, NCF)), full((1, NCF))]
                + [full((NCF, NCF)), full((NCF, NCF)), full((1, NCF))] * 4
                + [full((NCF, F)), full((1, F))])
    out_specs = [bC, bC, bF, bF]
    return pl.pallas_call(
        _dense_body,
        out_shape=(jax.ShapeDtypeStruct((NP, NCF), jnp.float32),
                   jax.ShapeDtypeStruct((NP, NCF), jnp.float32),
                   jax.ShapeDtypeStruct((NP, F), jnp.float32),
                   jax.ShapeDtypeStruct((NP, F), jnp.float32)),
        grid=(16,),
        in_specs=in_specs,
        out_specs=out_specs,
    )


_dense = _mk_dense()


def _score_body(wu, hi, out):
    s = jnp.sum(wu[:] * hi[:], axis=1, keepdims=True)
    out[:] = R_MIN + R_RANGE * jax.nn.sigmoid(s)


_score = pl.pallas_call(
    _score_body,
    out_shape=jax.ShapeDtypeStruct((B, 1), jnp.float32),
    grid=(8,),
    in_specs=[pl.BlockSpec((B // 8, F), lambda i: (i, 0))] * 2,
    out_specs=pl.BlockSpec((B // 8, 1), lambda i: (i, 0)),
)


def kernel(W0, H0, params, edge_index_user, edge_index_item, user_id, item_id):
    p = params
    f32 = jnp.float32

    def pad_edges(ei):
        src = jnp.concatenate([ei[0], jnp.zeros((EP - E,), jnp.int32)])
        dst = jnp.concatenate([ei[1], jnp.full((EP - E,), N, jnp.int32)])
        return src.reshape(EP // 128, 128), dst.reshape(EP // 128, 128)

    src_u, dst_u = pad_edges(edge_index_user)
    src_m, dst_m = pad_edges(edge_index_item)
    zeros = jnp.zeros((NP, F), f32)
    ones = jnp.ones((NP, F), f32)
    Wp = jnp.pad(W0, ((0, NP - N), (0, F - RANK)))
    Hp = jnp.pad(H0, ((0, NP - N), (0, F - RANK)))

    fl = lambda x: x.reshape(FLAT)
    fl2 = lambda x: x.reshape(_ACC_SHAPE)
    un = lambda x: x.reshape(NP, F)

    def wargs(side):
        s = 'W' if side == 'u' else 'H'
        wc = p['W_conv_' + s].reshape(ORD, RANK, NCF)
        wc = jnp.pad(wc, ((0, 0), (0, F - RANK), (0, 0))).reshape(ORD * F, NCF)
        bc = p['b_conv_' + s].reshape(1, NCF)
        g = []
        for gate in ('f', 'i', 'o', 'c'):
            g += [p['W_%s_%s' % (gate, side)], p['U_%s_%s' % (gate, side)],
                  p['b_%s_%s' % (gate, side)].reshape(1, NCF)]
        wout = jnp.pad(p['W_out_' + ('u' if side == 'u' else 'm')],
                       ((0, 0), (0, F - RANK)))
        bout = jnp.pad(p['b_out_' + ('u' if side == 'u' else 'm')],
                       (0, F - RANK)).reshape(1, F)
        return [wc, bc] + g + [wout, bout]

    wargs_u = wargs('u')
    wargs_m = wargs('m')

    deg_u, deg_m = _spmm(ones, ones, src_u, dst_u, src_m, dst_m, zeros)
    dv_u, xs_u, dv_m, xs_m = _prep(fl2(deg_u), fl2(deg_m), fl(Wp), fl(Hp))
    dv_u, dv_m = un(dv_u), un(dv_m)
    xs_u, xs_m = un(xs_u), un(xs_m)

    W16, H16 = Wp, Hp
    h_u = jnp.zeros((NP, NCF), f32)
    c_u = h_u
    h_m = h_u
    c_m = h_u

    for _ in range(NUM_IT):
        ts_u = [W16]
        ts_m = [H16]
        acc_u, acc_m = _spmm(xs_u, xs_m, src_u, dst_u, src_m, dst_m, zeros)
        t_u, xs_u, t_m, xs_m = _glue1(fl2(acc_u), fl(dv_u),
                                      fl2(acc_m), fl(dv_m))
        ts_u.append(un(t_u))
        ts_m.append(un(t_m))
        for _k in range(ORD - 2):
            acc_u, acc_m = _spmm(un(xs_u), un(xs_m),
                                 src_u, dst_u, src_m, dst_m, zeros)
            t_u, xs_u, t_m, xs_m = _glue2(fl2(acc_u), fl(ts_u[-2]), fl(dv_u),
                                          fl2(acc_m), fl(ts_m[-2]), fl(dv_m))
            ts_u.append(un(t_u))
            ts_m.append(un(t_m))
        h_u, c_u, W16, xs_u = _dense(*ts_u, h_u, c_u, W16, dv_u, *wargs_u)
        h_m, c_m, H16, xs_m = _dense(*ts_m, h_m, c_m, H16, dv_m, *wargs_m)

    uid2 = user_id.reshape(B // 128, 128)
    iid2 = item_id.reshape(B // 128, 128)
    wu, hi = _gather(W16, H16, uid2, iid2)
    score = _score(wu, hi)
    return score[:, 0]
